# Initial kernel scaffold; baseline (speedup 1.0000x reference)
import jax
import jax.numpy as jnp
from jax.experimental import pallas as pl


def kernel(x, W_embed, W_lin, b_lin, W1, b1, W2, b2, W3, b3):
    raise NotImplementedError("write your pallas kernel here")


# speedup vs baseline: 7.1681x; 7.1681x over previous
import functools

import jax
import jax.numpy as jnp
from jax import lax
from jax.experimental import pallas as pl
from jax.experimental.pallas import tpu as pltpu
from jax.experimental.pallas import tpu_sc as plsc

B = 16384
F = 26
D = 16
VOCAB_PER_FIELD = 100000
V = F * VOCAB_PER_FIELD
MLP_IN = F * D
H1 = 256
H2 = 128

BF = B * F
GW = 128
NROWS = BF // GW
NC, NS = 2, 16
NW = NC * NS
ROWS_PER_W = NROWS // NW
PIPE = 8
NGROUPS = ROWS_PER_W // PIPE

BATCH_BLK = 512
GRID = B // BATCH_BLK


def _sc_gather_body(idx_hbm, table_hbm, lin_hbm, emb_out, lin_out,
                    idx_v, rows_v, lin_v, sem_e, sem_l):
    wid = lax.axis_index("s") * NC + lax.axis_index("c")
    base = wid * ROWS_PER_W
    pltpu.sync_copy(idx_hbm.at[pl.ds(base, ROWS_PER_W)], idx_v)
    for g in range(NGROUPS):
        handles = []
        for j in range(PIPE):
            r = g * PIPE + j
            handles.append(
                pltpu.async_copy(table_hbm.at[idx_v.at[r]], rows_v.at[j], sem_e))
            handles.append(
                pltpu.async_copy(lin_hbm.at[idx_v.at[r]], lin_v.at[j], sem_l))
        for h in handles:
            h.wait()
        pltpu.sync_copy(rows_v, emb_out.at[pl.ds(base + g * PIPE, PIPE)])
        pltpu.sync_copy(lin_v, lin_out.at[pl.ds(base + g * PIPE, PIPE)])


_sc_gather = functools.partial(
    pl.kernel,
    out_type=(
        jax.ShapeDtypeStruct((NROWS, GW, D), jnp.float32),
        jax.ShapeDtypeStruct((NROWS, GW), jnp.float32),
    ),
    mesh=plsc.VectorSubcoreMesh(
        core_axis_name="c", subcore_axis_name="s", num_cores=NC,
        num_subcores=NS),
    compiler_params=pltpu.CompilerParams(use_tc_tiling_on_sc=False),
    scratch_types=(
        pltpu.VMEM((ROWS_PER_W, GW), jnp.int32),
        pltpu.VMEM((PIPE, GW, D), jnp.float32),
        pltpu.VMEM((PIPE, GW), jnp.float32),
        pltpu.SemaphoreType.DMA,
        pltpu.SemaphoreType.DMA,
    ),
)(_sc_gather_body)


def _mlp_body(emb_ref, lin_ref, w1_ref, b1_ref, w2_ref, b2_ref, w3_ref,
              c0_ref, out_ref):
    h = jnp.dot(emb_ref[...], w1_ref[...], preferred_element_type=jnp.float32)
    h = jnp.maximum(h + b1_ref[...], 0.0)
    h = jnp.dot(h, w2_ref[...], preferred_element_type=jnp.float32)
    h = jnp.maximum(h + b2_ref[...], 0.0)
    z = jnp.dot(h, w3_ref[...], preferred_element_type=jnp.float32)
    z = z + c0_ref[...] + jnp.sum(lin_ref[...], axis=1, keepdims=True)
    out_ref[...] = jax.nn.sigmoid(z)


_mlp = pl.pallas_call(
    _mlp_body,
    grid=(GRID,),
    in_specs=[
        pl.BlockSpec((BATCH_BLK, MLP_IN), lambda i: (i, 0)),
        pl.BlockSpec((BATCH_BLK, F), lambda i: (i, 0)),
        pl.BlockSpec((MLP_IN, H1), lambda i: (0, 0)),
        pl.BlockSpec((1, H1), lambda i: (0, 0)),
        pl.BlockSpec((H1, H2), lambda i: (0, 0)),
        pl.BlockSpec((1, H2), lambda i: (0, 0)),
        pl.BlockSpec((H2, 1), lambda i: (0, 0)),
        pl.BlockSpec((1, 1), lambda i: (0, 0)),
    ],
    out_specs=pl.BlockSpec((BATCH_BLK, 1), lambda i: (i, 0)),
    out_shape=jax.ShapeDtypeStruct((B, 1), jnp.float32),
)


def kernel(x, W_embed, W_lin, b_lin, W1, b1, W2, b2, W3, b3):
    offsets = (jnp.arange(F, dtype=jnp.int32) * VOCAB_PER_FIELD)
    idx = (x.astype(jnp.int32) + offsets[None, :]).reshape(NROWS, GW)
    emb, lin = _sc_gather(idx, W_embed, W_lin.reshape(V), *())
    emb2 = emb.reshape(B, MLP_IN)
    lin2 = lin.reshape(B, F)
    c0 = (b3 + b_lin).reshape(1, 1)
    out = _mlp(emb2, lin2, W1, b1.reshape(1, H1), W2, b2.reshape(1, H2),
               W3, c0)
    return out.reshape(B)
